# Initial kernel scaffold; baseline (speedup 1.0000x reference)
import jax
import jax.numpy as jnp
from jax.experimental import pallas as pl


def kernel(logits, k):
    raise NotImplementedError("write your pallas kernel here")


# speedup vs baseline: 16.4267x; 16.4267x over previous
import functools

import jax
import jax.numpy as jnp
import numpy as np
from jax import lax
from jax.experimental import pallas as pl
from jax.experimental.pallas import tpu as pltpu

_TEMP = 0.5
_TOP_P = 0.9
_NEG_BIG = -1e10
_POOL = 64
_INT_MAX = 2147483647


def _shift_left_lanes(x, s):
    return jnp.concatenate([x[:, s:], x[:, :s]], axis=1)


def _shift_right_lanes(x, s):
    return jnp.concatenate([x[:, -s:], x[:, :-s]], axis=1)


def _bitonic_merge_desc(arr, lane):
    for d in (64, 32, 16, 8, 4, 2, 1):
        up = _shift_left_lanes(arr, d)
        down = _shift_right_lanes(arr, d)
        take_max = (lane & d) == 0
        arr = jnp.where(take_max, jnp.maximum(arr, up), jnp.minimum(arr, down))
    return arr


def _pass1_kernel(V, C, k_ref, x_ref, params_ref, pool_ref, xf_ref, newv_ref):
    j = pl.program_id(0)
    R = x_ref.shape[0]
    lane = lax.broadcasted_iota(jnp.int32, (R, 128), 1)

    @pl.when(j == 0)
    def _():
        pool_ref[...] = jnp.full((R, 128), -jnp.inf, jnp.float32)

    col = j * C + lax.broadcasted_iota(jnp.int32, (R, C), 1)
    y = x_ref[...] * 2.0
    xf_ref[...] = jnp.where(col < V, y, -jnp.inf)
    newv_ref[...] = jnp.full((R, 128), -jnp.inf, jnp.float32)

    def body(i, carry):
        xf = xf_ref[...]
        m = jnp.max(xf, axis=1, keepdims=True)
        eq = xf == m
        candcol = jnp.where(eq, col, _INT_MAX)
        cstar = jnp.min(candcol, axis=1, keepdims=True)
        xf_ref[...] = jnp.where(eq & (candcol == cstar), -jnp.inf, xf)
        nv = newv_ref[...]
        newv_ref[...] = jnp.where(lane == (127 - i), m, nv)
        return carry

    lax.fori_loop(0, _POOL, body, 0)

    arr = jnp.where(lane < _POOL, pool_ref[...], newv_ref[...])
    arr = _bitonic_merge_desc(arr, lane)
    pool_ref[...] = arr

    svals = jnp.where(lane < _POOL, arr, -jnp.inf)
    kk = k_ref[0, 0]
    t_k = jnp.max(jnp.where(lane == kk - 1, svals, -jnp.inf), axis=1,
                  keepdims=True)
    m_row = jnp.max(svals, axis=1, keepdims=True)
    kept_k = svals >= t_k
    e = jnp.exp(svals - m_row)
    ek = jnp.where(kept_k, e, 0.0)
    z1 = jnp.sum(ek, axis=1, keepdims=True)
    ps = ek / z1
    cum = ps
    for s in (1, 2, 4, 8, 16, 32, 64):
        sh = _shift_right_lanes(cum, s)
        cum = cum + jnp.where(lane >= s, sh, 0.0)
    mask_p = (cum - ps) > _TOP_P
    sorted_logits = jnp.where(kept_k, svals, _NEG_BIG)
    kept_vals = jnp.where(mask_p, jnp.inf, sorted_logits)
    thresh = jnp.min(kept_vals, axis=1, keepdims=True)
    kept_final = svals >= thresh
    z2 = jnp.sum(jnp.where(kept_final, e, 0.0), axis=1, keepdims=True)
    params_ref[...] = jnp.where(
        lane == 0, thresh,
        jnp.where(lane == 1, m_row, jnp.where(lane == 2, z2, 0.0)))


def _rotl(v, r):
    return (v << r) | lax.shift_right_logical(v, 32 - r)


def _threefry_bits(x0, x1):
    ks0 = np.int32(0)
    ks1 = np.int32(1)
    ks2 = np.int32(0x1BD11BDB)
    x0 = x0 + ks0
    x1 = x1 + ks1
    rots = ((13, 15, 26, 6), (17, 29, 16, 24),
            (13, 15, 26, 6), (17, 29, 16, 24), (13, 15, 26, 6))
    inject = ((ks1, ks2 + 1), (ks2, ks0 + 2), (ks0, ks1 + 3),
              (ks1, ks2 + 4), (ks2, ks0 + 5))
    for g in range(5):
        for r in rots[g]:
            x0 = x0 + x1
            x1 = _rotl(x1, r)
            x1 = x1 ^ x0
        a, b = inject[g]
        x0 = x0 + a
        x1 = x1 + b
    return x0, x1


def _pass2_kernel(V, C, x_ref, params_ref, probs_ref, nc_ref, bv_ref, bc_ref):
    j = pl.program_id(0)
    nj = pl.num_programs(0)
    R = x_ref.shape[0]

    col = j * C + lax.broadcasted_iota(jnp.int32, (R, C), 1)
    y = x_ref[...] * 2.0
    p = params_ref[...]
    thresh = p[:, 0:1]
    m_row = p[:, 1:2]
    z = p[:, 2:3]
    kept = y >= thresh
    e = jnp.exp(y - m_row)
    probs = jnp.where(kept, e / z, 0.0)
    probs_ref[...] = probs

    flat = lax.broadcasted_iota(jnp.int32, (R, C), 0) * V + col
    b0, b1 = _threefry_bits(jnp.zeros_like(flat), flat)
    bits = b0 ^ b1
    ubits = lax.shift_right_logical(bits, 9) | np.int32(0x3F800000)
    u = lax.bitcast_convert_type(ubits, jnp.float32) - 1.0
    u = jnp.maximum(u, 0.0)
    g = -jnp.log(-jnp.log(u + 1e-10) + 1e-10)

    score = jnp.log(probs + 1e-10) + g
    score = jnp.where(col < V, score, -jnp.inf)
    bmax = jnp.max(score, axis=1, keepdims=True)
    candc = jnp.where(score == bmax, col, _INT_MAX)
    barg = jnp.min(candc, axis=1, keepdims=True)
    lane = lax.broadcasted_iota(jnp.int32, (R, 128), 1)
    bmax_b = jnp.broadcast_to(bmax, (R, 128))
    barg_b = jnp.broadcast_to(barg, (R, 128))

    @pl.when(j == 0)
    def _():
        bv_ref[...] = bmax_b
        bc_ref[...] = barg_b

    @pl.when(j > 0)
    def _():
        old_v = bv_ref[...]
        old_c = bc_ref[...]
        new_win = bmax_b > old_v
        bv_ref[...] = jnp.where(new_win, bmax_b, old_v)
        bc_ref[...] = jnp.where(new_win, barg_b, old_c)

    @pl.when(j == nj - 1)
    def _():
        nc_ref[...] = bc_ref[...]


_CHUNK1 = 16384
_CHUNK2 = 16384


def kernel(logits, k):
    R, V = logits.shape
    C1 = _CHUNK1
    C2 = _CHUNK2
    g1 = pl.cdiv(V, C1)
    g2 = pl.cdiv(V, C2)
    k_arr = jnp.asarray(k, jnp.int32).reshape(1, 1)

    params = pl.pallas_call(
        functools.partial(_pass1_kernel, V, C1),
        grid=(g1,),
        in_specs=[
            pl.BlockSpec((1, 1), lambda j: (0, 0),
                         memory_space=pltpu.SMEM),
            pl.BlockSpec((R, C1), lambda j: (0, j)),
        ],
        out_specs=pl.BlockSpec((R, 128), lambda j: (0, 0)),
        out_shape=jax.ShapeDtypeStruct((R, 128), jnp.float32),
        scratch_shapes=[
            pltpu.VMEM((R, 128), jnp.float32),
            pltpu.VMEM((R, C1), jnp.float32),
            pltpu.VMEM((R, 128), jnp.float32),
        ],
    )(k_arr, logits)

    probs, nc = pl.pallas_call(
        functools.partial(_pass2_kernel, V, C2),
        grid=(g2,),
        in_specs=[
            pl.BlockSpec((R, C2), lambda j: (0, j)),
            pl.BlockSpec((R, 128), lambda j: (0, 0)),
        ],
        out_specs=[
            pl.BlockSpec((R, C2), lambda j: (0, j)),
            pl.BlockSpec((R, 128), lambda j: (0, 0)),
        ],
        out_shape=[
            jax.ShapeDtypeStruct((R, V), jnp.float32),
            jax.ShapeDtypeStruct((R, 128), jnp.int32),
        ],
        scratch_shapes=[
            pltpu.VMEM((R, 128), jnp.float32),
            pltpu.VMEM((R, 128), jnp.int32),
        ],
    )(logits, params)

    next_char = nc[:, 0]
    return probs, next_char


# speedup vs baseline: 52.1424x; 3.1743x over previous
import functools

import jax
import jax.numpy as jnp
import numpy as np
from jax import lax
from jax.experimental import pallas as pl
from jax.experimental.pallas import tpu as pltpu

_TEMP = 0.5
_TOP_P = 0.9
_NEG_BIG = -1e10
_POOL = 64
_INT_MAX = 2147483647


def _shift_left_lanes(x, s):
    return jnp.concatenate([x[:, s:], x[:, :s]], axis=1)


def _shift_right_lanes(x, s):
    return jnp.concatenate([x[:, -s:], x[:, :-s]], axis=1)


def _bitonic_merge_desc(arr, lane):
    for d in (64, 32, 16, 8, 4, 2, 1):
        up = _shift_left_lanes(arr, d)
        down = _shift_right_lanes(arr, d)
        take_max = (lane & d) == 0
        arr = jnp.where(take_max, jnp.maximum(arr, up), jnp.minimum(arr, down))
    return arr


def _pass1_kernel(V, C, k_ref, x_ref, params_ref, pool_ref, xf_ref, newv_ref):
    j = pl.program_id(0)
    R = x_ref.shape[0]
    lane = lax.broadcasted_iota(jnp.int32, (R, 128), 1)

    @pl.when(j == 0)
    def _():
        pool_ref[...] = jnp.full((R, 128), -jnp.inf, jnp.float32)

    col = j * C + lax.broadcasted_iota(jnp.int32, (R, C), 1)
    y = x_ref[...] * 2.0
    t64 = jnp.max(jnp.where(lane == (_POOL - 1), pool_ref[...], -jnp.inf),
                  axis=1, keepdims=True)
    xf = jnp.where((col < V) & (y > t64), y, -jnp.inf)
    xf_ref[...] = xf
    newv_ref[...] = jnp.full((R, 128), -jnp.inf, jnp.float32)
    n_max = jnp.max(jnp.sum((xf != -jnp.inf).astype(jnp.int32), axis=1))

    def body(i, carry):
        @pl.when(i < n_max)
        def _():
            xf = xf_ref[...]
            m = jnp.max(xf, axis=1, keepdims=True)
            eq = xf == m
            candcol = jnp.where(eq, col, _INT_MAX)
            cstar = jnp.min(candcol, axis=1, keepdims=True)
            xf_ref[...] = jnp.where(eq & (candcol == cstar), -jnp.inf, xf)
            nv = newv_ref[...]
            newv_ref[...] = jnp.where(lane == (127 - i), m, nv)
        return carry

    lax.fori_loop(0, _POOL, body, 0)

    @pl.when(n_max > 0)
    def _():
        arr = jnp.where(lane < _POOL, pool_ref[...], newv_ref[...])
        pool_ref[...] = _bitonic_merge_desc(arr, lane)

    @pl.when(j == pl.num_programs(0) - 1)
    def _():
        _decision_stage(k_ref, pool_ref, params_ref, lane)


def _decision_stage(k_ref, pool_ref, params_ref, lane):
    svals = jnp.where(lane < _POOL, pool_ref[...], -jnp.inf)
    kk = k_ref[0, 0]
    t_k = jnp.max(jnp.where(lane == kk - 1, svals, -jnp.inf), axis=1,
                  keepdims=True)
    m_row = jnp.max(svals, axis=1, keepdims=True)
    kept_k = svals >= t_k
    e = jnp.exp(svals - m_row)
    ek = jnp.where(kept_k, e, 0.0)
    z1 = jnp.sum(ek, axis=1, keepdims=True)
    ps = ek / z1
    cum = ps
    for s in (1, 2, 4, 8, 16, 32, 64):
        sh = _shift_right_lanes(cum, s)
        cum = cum + jnp.where(lane >= s, sh, 0.0)
    mask_p = (cum - ps) > _TOP_P
    sorted_logits = jnp.where(kept_k, svals, _NEG_BIG)
    kept_vals = jnp.where(mask_p, jnp.inf, sorted_logits)
    thresh = jnp.min(kept_vals, axis=1, keepdims=True)
    kept_final = svals >= thresh
    z2 = jnp.sum(jnp.where(kept_final, e, 0.0), axis=1, keepdims=True)
    params_ref[...] = jnp.where(
        lane == 0, thresh,
        jnp.where(lane == 1, m_row, jnp.where(lane == 2, z2, 0.0)))


def _rotl(v, r):
    return (v << r) | lax.shift_right_logical(v, 32 - r)


def _threefry_bits(x0, x1):
    ks0 = np.int32(0)
    ks1 = np.int32(1)
    ks2 = np.int32(0x1BD11BDB)
    x0 = x0 + ks0
    x1 = x1 + ks1
    rots = ((13, 15, 26, 6), (17, 29, 16, 24),
            (13, 15, 26, 6), (17, 29, 16, 24), (13, 15, 26, 6))
    inject = ((ks1, ks2 + 1), (ks2, ks0 + 2), (ks0, ks1 + 3),
              (ks1, ks2 + 4), (ks2, ks0 + 5))
    for g in range(5):
        for r in rots[g]:
            x0 = x0 + x1
            x1 = _rotl(x1, r)
            x1 = x1 ^ x0
        a, b = inject[g]
        x0 = x0 + a
        x1 = x1 + b
    return x0, x1


def _pass2_kernel(V, C, x_ref, params_ref, probs_ref, nc_ref, bv_ref, bc_ref):
    j = pl.program_id(0)
    nj = pl.num_programs(0)
    R = x_ref.shape[0]

    col = j * C + lax.broadcasted_iota(jnp.int32, (R, C), 1)
    y = x_ref[...] * 2.0
    p = params_ref[...]
    thresh = p[:, 0:1]
    m_row = p[:, 1:2]
    z = p[:, 2:3]
    kept = y >= thresh
    e = jnp.exp(y - m_row)
    probs = jnp.where(kept, e / z, 0.0)
    probs_ref[...] = probs

    flat = lax.broadcasted_iota(jnp.int32, (R, C), 0) * V + col
    b0, b1 = _threefry_bits(jnp.zeros_like(flat), flat)
    bits = b0 ^ b1
    ubits = lax.shift_right_logical(bits, 9) | np.int32(0x3F800000)
    u = lax.bitcast_convert_type(ubits, jnp.float32) - 1.0
    u = jnp.maximum(u, 0.0)
    g = -jnp.log(-jnp.log(u + 1e-10) + 1e-10)

    score = jnp.log(probs + 1e-10) + g
    score = jnp.where(col < V, score, -jnp.inf)
    bmax = jnp.max(score, axis=1, keepdims=True)
    candc = jnp.where(score == bmax, col, _INT_MAX)
    barg = jnp.min(candc, axis=1, keepdims=True)
    lane = lax.broadcasted_iota(jnp.int32, (R, 128), 1)
    bmax_b = jnp.broadcast_to(bmax, (R, 128))
    barg_b = jnp.broadcast_to(barg, (R, 128))

    @pl.when(j == 0)
    def _():
        bv_ref[...] = bmax_b
        bc_ref[...] = barg_b

    @pl.when(j > 0)
    def _():
        old_v = bv_ref[...]
        old_c = bc_ref[...]
        new_win = bmax_b > old_v
        bv_ref[...] = jnp.where(new_win, bmax_b, old_v)
        bc_ref[...] = jnp.where(new_win, barg_b, old_c)

    @pl.when(j == nj - 1)
    def _():
        nc_ref[...] = bc_ref[...]


_CHUNK1 = 4096
_CHUNK2 = 16384


def kernel(logits, k):
    R, V = logits.shape
    C1 = _CHUNK1
    C2 = _CHUNK2
    g1 = pl.cdiv(V, C1)
    g2 = pl.cdiv(V, C2)
    k_arr = jnp.asarray(k, jnp.int32).reshape(1, 1)

    params = pl.pallas_call(
        functools.partial(_pass1_kernel, V, C1),
        grid=(g1,),
        in_specs=[
            pl.BlockSpec((1, 1), lambda j: (0, 0),
                         memory_space=pltpu.SMEM),
            pl.BlockSpec((R, C1), lambda j: (0, j)),
        ],
        out_specs=pl.BlockSpec((R, 128), lambda j: (0, 0)),
        out_shape=jax.ShapeDtypeStruct((R, 128), jnp.float32),
        scratch_shapes=[
            pltpu.VMEM((R, 128), jnp.float32),
            pltpu.VMEM((R, C1), jnp.float32),
            pltpu.VMEM((R, 128), jnp.float32),
        ],
    )(k_arr, logits)

    probs, nc = pl.pallas_call(
        functools.partial(_pass2_kernel, V, C2),
        grid=(g2,),
        in_specs=[
            pl.BlockSpec((R, C2), lambda j: (0, j)),
            pl.BlockSpec((R, 128), lambda j: (0, 0)),
        ],
        out_specs=[
            pl.BlockSpec((R, C2), lambda j: (0, j)),
            pl.BlockSpec((R, 128), lambda j: (0, 0)),
        ],
        out_shape=[
            jax.ShapeDtypeStruct((R, V), jnp.float32),
            jax.ShapeDtypeStruct((R, 128), jnp.int32),
        ],
        scratch_shapes=[
            pltpu.VMEM((R, 128), jnp.float32),
            pltpu.VMEM((R, 128), jnp.int32),
        ],
    )(logits, params)

    next_char = nc[:, 0]
    return probs, next_char
